# baseline (speedup 1.0000x reference)
import functools

import jax
import jax.numpy as jnp
import numpy as np
from jax import lax
from jax.experimental import pallas as pl
from jax.experimental.pallas import tpu as pltpu
from jax.experimental.pallas import tpu_sc as plsc

N = 10000
D = 256
DO = 128
E = 160000
NT = 32
RPT = 320
NPAD = NT * RPT
EB = 128
ECH = 1024
EPAD = E + ECH
NPOW = 50

def _grid_positions():
    g = []
    for h in (0, 1):
        base = 80000 * h
        for k in range(1, 12):
            g.append(base + 5040 * k)
        for j in range(1, 5):
            g.append(base + 55440 + 4928 * j)
    g.append(80000)
    return sorted(set(p for p in g if 0 < p < E))

GRID = _grid_positions()
NB = len(GRID)


def _head_body(a_ref, w_ref, b_ref, o_ref):
    o_ref[...] = jnp.dot(a_ref[...], w_ref[...],
                         preferred_element_type=jnp.float32) + b_ref[...]


def _getscal(vref, idx, lanes):
    base = (idx // 16) * 16
    vec = vref[pl.ds(base, 16)]
    lane = idx - base
    return jnp.max(jnp.where(lanes == lane, vec, 0))


def _sc_iter_body(v_hbm, xc_hbm, src_hbm, dst_hbm, w_hbm, pk_hbm,
                  out_hbm,
                  acc, carry, rows, srcc, dstc, wc, pkv):
    cid = lax.axis_index("c")
    sid = lax.axis_index("s")
    wid = sid * 2 + cid
    row_base = wid * RPT
    lanes = lax.iota(jnp.int32, 16)
    zero16 = jnp.zeros((16,), jnp.float32)

    @pl.loop(0, RPT + 1)
    def _(r):
        for cg in range(D // 16):
            acc[r, pl.ds(cg * 16, 16)] = zero16

    @pl.loop(0, 33)
    def _(r):
        for cg in range(D // 16):
            carry[r, pl.ds(cg * 16, 16)] = zero16

    pltpu.sync_copy(pk_hbm.at[wid], pkv)
    nseg = _getscal(pkv, 126, lanes)

    def seg_body(s, c0):
        seg0 = _getscal(pkv, s, lanes)
        seg1 = _getscal(pkv, s + 1, lanes)
        astart = (seg0 // 16) * 16
        nch = (seg1 - astart + ECH - 1) // ECH

        def chunk_body(ci, c1):
            cb = astart + ci * ECH
            pltpu.sync_copy(src_hbm.at[pl.ds(cb, ECH)], srcc)
            pltpu.sync_copy(dst_hbm.at[pl.ds(cb, ECH)], dstc)
            pltpu.sync_copy(w_hbm.at[pl.ds(cb, ECH)], wc)
            nb = jnp.minimum(ECH // EB, (seg1 - cb + EB - 1) // EB)

            def batch_body(bi, c2):
                off0 = bi * EB
                pltpu.sync_copy(v_hbm.at[srcc.at[pl.ds(off0, EB)]], rows)
                groups = []
                for g in range(EB // 16):
                    off = off0 + g * 16
                    w16 = wc[pl.ds(off, 16)]
                    e16 = cb + off + lanes
                    ok = (e16 >= seg0) & (e16 < seg1)
                    w16 = jnp.where(ok, w16, 0.0)
                    d16 = dstc[pl.ds(off, 16)] - row_base
                    d16 = jnp.clip(d16, 0, RPT - 1)
                    j16 = lanes + g * 16
                    groups.append((w16, d16, j16))

                def col_body(c, c3):
                    c16 = jnp.zeros((16,), jnp.int32) + c
                    for (w16, d16, j16) in groups:
                        vals = plsc.load_gather(rows, [j16, c16])
                        plsc.addupdate_scatter(acc, [d16, c16], w16 * vals)
                    return c3

                lax.fori_loop(0, D, col_body, 0)
                return c2

            lax.fori_loop(0, nb, batch_body, 0)
            return c1

        lax.fori_loop(0, nch, chunk_body, 0)

        @pl.when(s + 1 < nseg)
        def _():
            fr = _getscal(pkv, 33 + s, lanes)
            fs = _getscal(pkv, 64 + s, lanes)
            for cg in range(D // 16):
                sl = pl.ds(cg * 16, 16)
                carry[fs, sl] = carry[fs, sl] + acc[fr, sl]
                acc[fr, sl] = zero16

        return c0

    lax.fori_loop(0, nseg, seg_body, 0)

    def merge_body(m, c0):
        mr = _getscal(pkv, 95 + m, lanes)
        ms = _getscal(pkv, 64 + m, lanes)
        for cg in range(D // 16):
            sl = pl.ds(cg * 16, 16)
            acc[mr, sl] = carry[ms, sl] + acc[mr, sl]
        return c0

    lax.fori_loop(0, nseg - 1, merge_body, 0)

    for off, sz in ((0, 128), (128, 128), (256, RPT - 256)):
        pltpu.sync_copy(xc_hbm.at[pl.ds(row_base + off, sz)],
                        rows.at[pl.ds(0, sz)])

        def blend_body(r, c0, off=off):
            for cg in range(D // 16):
                sl = pl.ds(cg * 16, 16)
                rows[r, sl] = 0.5 * acc[off + r, sl] + 0.5 * rows[r, sl]
            return c0

        lax.fori_loop(0, sz, blend_body, 0)
        pltpu.sync_copy(rows.at[pl.ds(0, sz)],
                        out_hbm.at[pl.ds(row_base + off, sz)])


def _make_sc_call():
    mesh = plsc.VectorSubcoreMesh(core_axis_name="c", subcore_axis_name="s")
    return pl.kernel(
        _sc_iter_body,
        mesh=mesh,
        out_type=jax.ShapeDtypeStruct((NPAD, D), jnp.float32),
        scratch_types=[
            pltpu.VMEM((RPT + 1, D), jnp.float32),
            pltpu.VMEM((33, D), jnp.float32),
            pltpu.VMEM((EB, D), jnp.float32),
            pltpu.VMEM((ECH,), jnp.int32),
            pltpu.VMEM((ECH,), jnp.int32),
            pltpu.VMEM((ECH,), jnp.float32),
            pltpu.VMEM((128,), jnp.int32),
        ],
        compiler_params=pltpu.CompilerParams(needs_layout_passes=False),
    )


def _preprocess(src, dst, ew):
    order = jnp.argsort(dst)
    ssrc = src[order]
    sdst = dst[order]
    sw = ew[order]
    pad = EPAD - E
    ssrc_p = jnp.concatenate([ssrc, jnp.zeros((pad,), jnp.int32)])
    sdst_p = jnp.concatenate([sdst, jnp.full((pad,), N - 1, jnp.int32)])
    sw_p = jnp.concatenate([sw, jnp.zeros((pad,), jnp.float32)])

    rows32 = jnp.arange(NT, dtype=jnp.int32)
    tstart = jnp.searchsorted(sdst, rows32 * RPT).astype(jnp.int32)
    tend = jnp.searchsorted(sdst, (rows32 + 1) * RPT).astype(jnp.int32)

    gridj = jnp.asarray(GRID, jnp.int32)
    grow = sdst[gridj]
    gtile = grow // RPT
    growl = grow - gtile * RPT

    is_t = gtile[None, :] == rows32[:, None]
    sentinel = jnp.int32(EPAD + 8)
    brk_raw = jnp.where(is_t, gridj[None, :], sentinel)
    ordb = jnp.argsort(brk_raw, axis=1)
    brk = jnp.take_along_axis(brk_raw, ordb, axis=1)
    frow_raw = jnp.where(is_t, growl[None, :], jnp.int32(RPT))
    frow = jnp.take_along_axis(frow_raw, ordb, axis=1)
    nbrk = is_t.sum(axis=1).astype(jnp.int32)
    nseg = nbrk + 1

    segs_mid = jnp.minimum(brk, tend[:, None])
    segs = jnp.concatenate([tstart[:, None], segs_mid, tend[:, None]], axis=1)

    idxs = jnp.arange(NB, dtype=jnp.int32)
    eq = (frow[:, :, None] == frow[:, None, :]) & \
         (idxs[None, None, :] <= idxs[None, :, None])
    fslot = jnp.argmax(eq, axis=2).astype(jnp.int32)
    valid = idxs[None, :] < nbrk[:, None]
    fslot = jnp.where(valid, fslot, jnp.int32(32))
    first_occ = fslot == idxs[None, :]
    mrow = jnp.where(valid & first_occ, frow, jnp.int32(RPT))

    pk = jnp.concatenate(
        [segs, frow, fslot, mrow,
         nseg[:, None], jnp.zeros((NT, 1), jnp.int32)], axis=1)
    return ssrc_p, sdst_p, sw_p, pk


def kernel(x, edge_index, edge_weight, weight, bias):
    x = x.astype(jnp.float32)
    ew = edge_weight.astype(jnp.float32)
    src = edge_index[1].astype(jnp.int32)
    dst = edge_index[0].astype(jnp.int32)

    ssrc, sdst, sw, pk = _preprocess(src, dst, ew)

    mean = jnp.sum(x, axis=0) * np.float32(1.0 / N)
    xc = x - mean[None, :]
    xcpad = jnp.pad(xc, ((0, NPAD - N), (0, 0)))

    sc_call = _make_sc_call()

    def body(_, v):
        return sc_call(v, xcpad, ssrc, sdst, sw, pk)

    v = lax.fori_loop(0, NPOW, body, xcpad)

    out = pl.pallas_call(
        _head_body,
        out_shape=jax.ShapeDtypeStruct((N, DO), jnp.float32),
    )(v[:N], weight.astype(jnp.float32), bias.astype(jnp.float32))
    return out


# speedup vs baseline: 4.8695x; 4.8695x over previous
import functools

import jax
import jax.numpy as jnp
import numpy as np
from jax import lax
from jax.experimental import pallas as pl
from jax.experimental.pallas import tpu as pltpu
from jax.experimental.pallas import tpu_sc as plsc

N = 10000
D = 256
DO = 128
E = 160000
NT = 32
RPT = 320
NPAD = NT * RPT
EB = 128
ECH = 512
EPAD = E + 1024
NPOW = 50

def _grid_positions():
    g = []
    for h in (0, 1):
        base = 80000 * h
        for k in range(1, 12):
            g.append(base + 5040 * k)
        for j in range(1, 5):
            g.append(base + 55440 + 4928 * j)
    g.append(80000)
    return sorted(set(p for p in g if 0 < p < E))

GRID = _grid_positions()
NB = len(GRID)


def _head_body(a_ref, w_ref, b_ref, o_ref):
    o_ref[...] = jnp.dot(a_ref[...], w_ref[...],
                         preferred_element_type=jnp.float32) + b_ref[...]


def _getscal(vref, idx, lanes):
    base = (idx // 16) * 16
    vec = vref[pl.ds(base, 16)]
    lane = idx - base
    return jnp.max(jnp.where(lanes == lane, vec, 0))


def _sc_iter_body(v_hbm, xc_hbm, src_hbm, dst_hbm, w_hbm, pk_hbm,
                  out_hbm,
                  acc, carry, rows, srcc, dstc, wc, pkv):
    cid = lax.axis_index("c")
    sid = lax.axis_index("s")
    wid = sid * 2 + cid
    row_base = wid * RPT
    lanes = lax.iota(jnp.int32, 16)
    zero16 = jnp.zeros((16,), jnp.float32)

    @pl.loop(0, RPT + 1)
    def _(r):
        for cg in range(D // 16):
            acc[r, pl.ds(cg * 16, 16)] = zero16

    @pl.loop(0, 33)
    def _(r):
        for cg in range(D // 16):
            carry[r, pl.ds(cg * 16, 16)] = zero16

    pltpu.sync_copy(pk_hbm.at[wid], pkv)
    nseg = _getscal(pkv, 126, lanes)

    def seg_body(s, c0):
        seg0 = _getscal(pkv, s, lanes)
        seg1 = _getscal(pkv, s + 1, lanes)
        astart = (seg0 // 16) * 16
        nch = (seg1 - astart + ECH - 1) // ECH

        def chunk_body(ci, c1):
            cb = astart + ci * ECH
            pltpu.sync_copy(src_hbm.at[pl.ds(cb, ECH)], srcc)
            pltpu.sync_copy(dst_hbm.at[pl.ds(cb, ECH)], dstc)
            pltpu.sync_copy(w_hbm.at[pl.ds(cb, ECH)], wc)
            nb = jnp.minimum(ECH // EB, (seg1 - cb + EB - 1) // EB)

            def batch_body(bi, c2):
                off0 = bi * EB
                pltpu.sync_copy(v_hbm.at[srcc.at[pl.ds(off0, EB)]], rows)
                lo = jnp.maximum(seg0 - cb, off0)
                hi = jnp.minimum(seg1 - cb, off0 + EB)

                def edge_body(el, c3):
                    j = el - off0
                    es = jnp.zeros((16,), jnp.int32) + el
                    dl16 = plsc.load_gather(dstc, [es]) - row_base
                    w16 = plsc.load_gather(wc, [es])
                    for cg in range(D // 16):
                        vals = rows[j, pl.ds(cg * 16, 16)]
                        plsc.addupdate_scatter(
                            acc, [dl16, lanes + cg * 16], w16 * vals)
                    return c3

                lax.fori_loop(lo, hi, edge_body, 0)
                return c2

            lax.fori_loop(0, nb, batch_body, 0)
            return c1

        lax.fori_loop(0, nch, chunk_body, 0)

        @pl.when(s + 1 < nseg)
        def _():
            fr = _getscal(pkv, 33 + s, lanes)
            fs = _getscal(pkv, 64 + s, lanes)
            for cg in range(D // 16):
                sl = pl.ds(cg * 16, 16)
                carry[fs, sl] = carry[fs, sl] + acc[fr, sl]
                acc[fr, sl] = zero16

        return c0

    lax.fori_loop(0, nseg, seg_body, 0)

    def merge_body(m, c0):
        mr = _getscal(pkv, 95 + m, lanes)
        ms = _getscal(pkv, 64 + m, lanes)
        for cg in range(D // 16):
            sl = pl.ds(cg * 16, 16)
            acc[mr, sl] = carry[ms, sl] + acc[mr, sl]
        return c0

    lax.fori_loop(0, nseg - 1, merge_body, 0)

    for off, sz in ((0, 128), (128, 128), (256, RPT - 256)):
        pltpu.sync_copy(xc_hbm.at[pl.ds(row_base + off, sz)],
                        rows.at[pl.ds(0, sz)])

        def blend_body(r, c0, off=off):
            for cg in range(D // 16):
                sl = pl.ds(cg * 16, 16)
                rows[r, sl] = 0.5 * acc[off + r, sl] + 0.5 * rows[r, sl]
            return c0

        lax.fori_loop(0, sz, blend_body, 0)
        pltpu.sync_copy(rows.at[pl.ds(0, sz)],
                        out_hbm.at[pl.ds(row_base + off, sz)])


def _make_sc_call():
    mesh = plsc.VectorSubcoreMesh(core_axis_name="c", subcore_axis_name="s")
    return pl.kernel(
        _sc_iter_body,
        mesh=mesh,
        out_type=jax.ShapeDtypeStruct((NPAD, D), jnp.float32),
        scratch_types=[
            pltpu.VMEM((RPT + 1, D), jnp.float32),
            pltpu.VMEM((33, D), jnp.float32),
            pltpu.VMEM((EB, D), jnp.float32),
            pltpu.VMEM((ECH,), jnp.int32),
            pltpu.VMEM((ECH,), jnp.int32),
            pltpu.VMEM((ECH,), jnp.float32),
            pltpu.VMEM((128,), jnp.int32),
        ],
        compiler_params=pltpu.CompilerParams(needs_layout_passes=False),
    )


def _preprocess(src, dst, ew):
    order = jnp.argsort(dst)
    ssrc = src[order]
    sdst = dst[order]
    sw = ew[order]
    pad = EPAD - E
    ssrc_p = jnp.concatenate([ssrc, jnp.zeros((pad,), jnp.int32)])
    sdst_p = jnp.concatenate([sdst, jnp.full((pad,), N - 1, jnp.int32)])
    sw_p = jnp.concatenate([sw, jnp.zeros((pad,), jnp.float32)])

    rows32 = jnp.arange(NT, dtype=jnp.int32)
    tstart = jnp.searchsorted(sdst, rows32 * RPT).astype(jnp.int32)
    tend = jnp.searchsorted(sdst, (rows32 + 1) * RPT).astype(jnp.int32)

    gridj = jnp.asarray(GRID, jnp.int32)
    grow = sdst[gridj]
    gtile = grow // RPT
    growl = grow - gtile * RPT

    is_t = gtile[None, :] == rows32[:, None]
    sentinel = jnp.int32(EPAD + 8)
    brk_raw = jnp.where(is_t, gridj[None, :], sentinel)
    ordb = jnp.argsort(brk_raw, axis=1)
    brk = jnp.take_along_axis(brk_raw, ordb, axis=1)
    frow_raw = jnp.where(is_t, growl[None, :], jnp.int32(RPT))
    frow = jnp.take_along_axis(frow_raw, ordb, axis=1)
    nbrk = is_t.sum(axis=1).astype(jnp.int32)
    nseg = nbrk + 1

    segs_mid = jnp.minimum(brk, tend[:, None])
    segs = jnp.concatenate([tstart[:, None], segs_mid, tend[:, None]], axis=1)

    idxs = jnp.arange(NB, dtype=jnp.int32)
    eq = (frow[:, :, None] == frow[:, None, :]) & \
         (idxs[None, None, :] <= idxs[None, :, None])
    fslot = jnp.argmax(eq, axis=2).astype(jnp.int32)
    valid = idxs[None, :] < nbrk[:, None]
    fslot = jnp.where(valid, fslot, jnp.int32(32))
    first_occ = fslot == idxs[None, :]
    mrow = jnp.where(valid & first_occ, frow, jnp.int32(RPT))

    pk = jnp.concatenate(
        [segs, frow, fslot, mrow,
         nseg[:, None], jnp.zeros((NT, 1), jnp.int32)], axis=1)
    return ssrc_p, sdst_p, sw_p, pk


def kernel(x, edge_index, edge_weight, weight, bias):
    x = x.astype(jnp.float32)
    ew = edge_weight.astype(jnp.float32)
    src = edge_index[1].astype(jnp.int32)
    dst = edge_index[0].astype(jnp.int32)

    ssrc, sdst, sw, pk = _preprocess(src, dst, ew)

    mean = jnp.sum(x, axis=0) * np.float32(1.0 / N)
    xc = x - mean[None, :]
    xcpad = jnp.pad(xc, ((0, NPAD - N), (0, 0)))

    sc_call = _make_sc_call()

    def body(_, v):
        return sc_call(v, xcpad, ssrc, sdst, sw, pk)

    v = lax.fori_loop(0, NPOW, body, xcpad)

    out = pl.pallas_call(
        _head_body,
        out_shape=jax.ShapeDtypeStruct((N, DO), jnp.float32),
    )(v[:N], weight.astype(jnp.float32), bias.astype(jnp.float32))
    return out


# speedup vs baseline: 4.9255x; 1.0115x over previous
import jax
import jax.numpy as jnp
import numpy as np
from jax import lax
from jax.experimental import pallas as pl
from jax.experimental.pallas import tpu as pltpu
from jax.experimental.pallas import tpu_sc as plsc

N = 10000
D = 256
DO = 128
E = 160000
NT = 32
RPT = 320
NPAD = NT * RPT
EB = 128
ECH = 1024
EPAD = E + 1024
NPOW = 50

def _grid_positions():
    g = []
    for h in (0, 1):
        base = 80000 * h
        for k in range(1, 12):
            g.append(base + 5040 * k)
        for j in range(1, 5):
            g.append(base + 55440 + 4928 * j)
    g.append(80000)
    return sorted(set(p for p in g if 0 < p < E))

GRID = _grid_positions()
NB = len(GRID)


def _head_body(a_ref, w_ref, b_ref, o_ref):
    o_ref[...] = jnp.dot(a_ref[...], w_ref[...],
                         preferred_element_type=jnp.float32) + b_ref[...]


def _getscal(vref, idx, lanes):
    base = (idx // 16) * 16
    vec = vref[pl.ds(base, 16)]
    lane = idx - base
    return jnp.max(jnp.where(lanes == lane, vec, 0))


def _sc_iter_body(v_hbm, xc_hbm, src_hbm, dst_hbm, w_hbm, pk_hbm,
                  out_hbm,
                  acc, carry, rows, srcc, dstc, wc, pkv):
    cid = lax.axis_index("c")
    sid = lax.axis_index("s")
    wid = sid * 2 + cid
    row_base = wid * RPT
    lanes = lax.iota(jnp.int32, 16)
    zero16 = jnp.zeros((16,), jnp.float32)

    @pl.loop(0, RPT + 1)
    def _(r):
        for cg in range(D // 16):
            acc[r, pl.ds(cg * 16, 16)] = zero16

    @pl.loop(0, 33)
    def _(r):
        for cg in range(D // 16):
            carry[r, pl.ds(cg * 16, 16)] = zero16

    pltpu.sync_copy(pk_hbm.at[wid], pkv)
    nseg = _getscal(pkv, 126, lanes)

    def seg_body(s, c0):
        seg0 = _getscal(pkv, s, lanes)
        seg1 = _getscal(pkv, s + 1, lanes)
        astart = (seg0 // 16) * 16
        nch = (seg1 - astart + ECH - 1) // ECH

        def chunk_body(ci, c1):
            cb = astart + ci * ECH
            pltpu.sync_copy(src_hbm.at[pl.ds(cb, ECH)], srcc)
            pltpu.sync_copy(dst_hbm.at[pl.ds(cb, ECH)], dstc)
            pltpu.sync_copy(w_hbm.at[pl.ds(cb, ECH)], wc)
            nb = jnp.minimum(ECH // EB, (seg1 - cb + EB - 1) // EB)

            def batch_body(bi, c2):
                off0 = bi * EB
                pltpu.sync_copy(v_hbm.at[srcc.at[pl.ds(off0, EB)]], rows)
                lo = jnp.maximum(seg0 - cb, off0)
                hi = jnp.minimum(seg1 - cb, off0 + EB)

                def edge_body(el, c3):
                    j = el - off0
                    es = jnp.zeros((16,), jnp.int32) + el
                    dl16 = plsc.load_gather(dstc, [es]) - row_base
                    w16 = plsc.load_gather(wc, [es])
                    for cg in range(D // 16):
                        vals = rows[j, pl.ds(cg * 16, 16)]
                        plsc.addupdate_scatter(
                            acc, [dl16, lanes + cg * 16], w16 * vals)
                    return c3

                lax.fori_loop(lo, hi, edge_body, 0)
                return c2

            lax.fori_loop(0, nb, batch_body, 0)
            return c1

        lax.fori_loop(0, nch, chunk_body, 0)

        @pl.when(s + 1 < nseg)
        def _():
            fr = _getscal(pkv, 33 + s, lanes)
            fs = _getscal(pkv, 64 + s, lanes)
            for cg in range(D // 16):
                sl = pl.ds(cg * 16, 16)
                carry[fs, sl] = carry[fs, sl] + acc[fr, sl]
                acc[fr, sl] = zero16

        return c0

    lax.fori_loop(0, nseg, seg_body, 0)

    def merge_body(m, c0):
        mr = _getscal(pkv, 95 + m, lanes)
        ms = _getscal(pkv, 64 + m, lanes)
        for cg in range(D // 16):
            sl = pl.ds(cg * 16, 16)
            acc[mr, sl] = carry[ms, sl] + acc[mr, sl]
        return c0

    lax.fori_loop(0, nseg - 1, merge_body, 0)

    for off, sz in ((0, 128), (128, 128), (256, RPT - 256)):
        pltpu.sync_copy(xc_hbm.at[pl.ds(row_base + off, sz)],
                        rows.at[pl.ds(0, sz)])

        def blend_body(r, c0, off=off):
            for cg in range(D // 16):
                sl = pl.ds(cg * 16, 16)
                rows[r, sl] = 0.5 * acc[off + r, sl] + 0.5 * rows[r, sl]
            return c0

        lax.fori_loop(0, sz, blend_body, 0)
        pltpu.sync_copy(rows.at[pl.ds(0, sz)],
                        out_hbm.at[pl.ds(row_base + off, sz)])


def _make_sc_call():
    mesh = plsc.VectorSubcoreMesh(core_axis_name="c", subcore_axis_name="s")
    return pl.kernel(
        _sc_iter_body,
        mesh=mesh,
        out_type=jax.ShapeDtypeStruct((NPAD, D), jnp.float32),
        scratch_types=[
            pltpu.VMEM((RPT + 1, D), jnp.float32),
            pltpu.VMEM((33, D), jnp.float32),
            pltpu.VMEM((EB, D), jnp.float32),
            pltpu.VMEM((ECH,), jnp.int32),
            pltpu.VMEM((ECH,), jnp.int32),
            pltpu.VMEM((ECH,), jnp.float32),
            pltpu.VMEM((128,), jnp.int32),
        ],
        compiler_params=pltpu.CompilerParams(needs_layout_passes=False),
    )


def _preprocess(src, dst, ew):
    order = jnp.argsort(dst)
    ssrc = src[order]
    sdst = dst[order]
    sw = ew[order]
    pad = EPAD - E
    ssrc_p = jnp.concatenate([ssrc, jnp.zeros((pad,), jnp.int32)])
    sdst_p = jnp.concatenate([sdst, jnp.full((pad,), N - 1, jnp.int32)])
    sw_p = jnp.concatenate([sw, jnp.zeros((pad,), jnp.float32)])

    rows32 = jnp.arange(NT, dtype=jnp.int32)
    tstart = jnp.searchsorted(sdst, rows32 * RPT).astype(jnp.int32)
    tend = jnp.searchsorted(sdst, (rows32 + 1) * RPT).astype(jnp.int32)

    gridj = jnp.asarray(GRID, jnp.int32)
    grow = sdst[gridj]
    gtile = grow // RPT
    growl = grow - gtile * RPT

    is_t = gtile[None, :] == rows32[:, None]
    sentinel = jnp.int32(EPAD + 8)
    brk_raw = jnp.where(is_t, gridj[None, :], sentinel)
    ordb = jnp.argsort(brk_raw, axis=1)
    brk = jnp.take_along_axis(brk_raw, ordb, axis=1)
    frow_raw = jnp.where(is_t, growl[None, :], jnp.int32(RPT))
    frow = jnp.take_along_axis(frow_raw, ordb, axis=1)
    nbrk = is_t.sum(axis=1).astype(jnp.int32)
    nseg = nbrk + 1

    segs_mid = jnp.minimum(brk, tend[:, None])
    segs = jnp.concatenate([tstart[:, None], segs_mid, tend[:, None]], axis=1)

    idxs = jnp.arange(NB, dtype=jnp.int32)
    eq = (frow[:, :, None] == frow[:, None, :]) & \
         (idxs[None, None, :] <= idxs[None, :, None])
    fslot = jnp.argmax(eq, axis=2).astype(jnp.int32)
    valid = idxs[None, :] < nbrk[:, None]
    fslot = jnp.where(valid, fslot, jnp.int32(32))
    first_occ = fslot == idxs[None, :]
    mrow = jnp.where(valid & first_occ, frow, jnp.int32(RPT))

    pk = jnp.concatenate(
        [segs, frow, fslot, mrow,
         nseg[:, None], jnp.zeros((NT, 1), jnp.int32)], axis=1)
    return ssrc_p, sdst_p, sw_p, pk


def kernel(x, edge_index, edge_weight, weight, bias):
    x = x.astype(jnp.float32)
    ew = edge_weight.astype(jnp.float32)
    src = edge_index[1].astype(jnp.int32)
    dst = edge_index[0].astype(jnp.int32)

    ssrc, sdst, sw, pk = _preprocess(src, dst, ew)

    mean = jnp.sum(x, axis=0) * np.float32(1.0 / N)
    xc = x - mean[None, :]
    xcpad = jnp.pad(xc, ((0, NPAD - N), (0, 0)))

    sc_call = _make_sc_call()

    def body(_, v):
        return sc_call(v, xcpad, ssrc, sdst, sw, pk)

    v = lax.fori_loop(0, NPOW, body, xcpad)

    out = pl.pallas_call(
        _head_body,
        out_shape=jax.ShapeDtypeStruct((N, DO), jnp.float32),
    )(v[:N], weight.astype(jnp.float32), bias.astype(jnp.float32))
    return out
